# Initial kernel scaffold; baseline (speedup 1.0000x reference)
import jax
import jax.numpy as jnp
from jax.experimental import pallas as pl


def kernel(x, router_logits, skill_gate, skill_up, skill_down, shared_gate, shared_up, shared_down):
    raise NotImplementedError("write your pallas kernel here")


# speedup vs baseline: 3.4423x; 3.4423x over previous
import functools

import jax
import jax.numpy as jnp
from jax.experimental import pallas as pl
from jax.experimental.pallas import tpu as pltpu


def _router_body(logits_ref, idx_ref, scl_ref):
    lg = logits_ref[...].astype(jnp.float32)
    m = jnp.max(lg, axis=-1, keepdims=True)
    e = jnp.exp(lg - m)
    p = e / jnp.sum(e, axis=-1, keepdims=True)
    B, E = p.shape
    cols = jax.lax.broadcasted_iota(jnp.int32, (B, E), 1)
    v1 = jnp.max(p, axis=-1)
    i1 = jnp.min(jnp.where(p == v1[:, None], cols, E), axis=-1)
    pm = jnp.where(cols == i1[:, None], -jnp.inf, p)
    v2 = jnp.max(pm, axis=-1)
    i2 = jnp.min(jnp.where(pm == v2[:, None], cols, E), axis=-1)
    tot = v1 + v2
    idx_ref[...] = jnp.stack([i1, i2], axis=-1).astype(jnp.int32)
    scl_ref[...] = jnp.stack([v1 / tot, v2 / tot], axis=-1)


def _route(router_logits):
    B, E = router_logits.shape
    return pl.pallas_call(
        _router_body,
        out_shape=(
            jax.ShapeDtypeStruct((B, 2), jnp.int32),
            jax.ShapeDtypeStruct((B, 2), jnp.float32),
        ),
    )(router_logits)


def _routed_body_weighted(nf, idx_ref, scl_ref, x_ref, g_ref, u_ref, d_ref,
                          o_ref, acc_ref):
    s, j, f = pl.program_id(0), pl.program_id(1), pl.program_id(2)

    @pl.when(jnp.logical_and(j == 0, f == 0))
    def _():
        acc_ref[...] = jnp.zeros_like(acc_ref)

    xb = x_ref[0]
    g = g_ref[0].astype(jnp.bfloat16)
    u = u_ref[0].astype(jnp.bfloat16)
    d = d_ref[0].astype(jnp.bfloat16)
    xg = jnp.dot(xb, g, preferred_element_type=jnp.float32)
    xu = jnp.dot(xb, u, preferred_element_type=jnp.float32)
    h = jax.nn.gelu(xg, approximate=True) * xu
    w = scl_ref[s, j]
    acc_ref[...] += w * jnp.dot(h.astype(jnp.bfloat16), d,
                                preferred_element_type=jnp.float32)

    @pl.when(jnp.logical_and(j == 1, f == nf - 1))
    def _():
        o_ref[0] = acc_ref[...]


def _routed(x16, idx, scl, skill_gate, skill_up, skill_down, fblk):
    B, T, D = x16.shape
    E, _, F = skill_gate.shape
    nf = F // fblk
    grid = (B, 2, nf)
    spec = pltpu.PrefetchScalarGridSpec(
        num_scalar_prefetch=2,
        grid=grid,
        in_specs=[
            pl.BlockSpec((1, T, D), lambda s, j, f, i_r, s_r: (s, 0, 0)),
            pl.BlockSpec((1, D, fblk),
                         lambda s, j, f, i_r, s_r: (i_r[s, j], 0, f)),
            pl.BlockSpec((1, D, fblk),
                         lambda s, j, f, i_r, s_r: (i_r[s, j], 0, f)),
            pl.BlockSpec((1, fblk, D),
                         lambda s, j, f, i_r, s_r: (i_r[s, j], f, 0)),
        ],
        out_specs=pl.BlockSpec((1, T, D), lambda s, j, f, i_r, s_r: (s, 0, 0)),
        scratch_shapes=[pltpu.VMEM((T, D), jnp.float32)],
    )
    return pl.pallas_call(
        functools.partial(_routed_body_weighted, nf),
        grid_spec=spec,
        out_shape=jax.ShapeDtypeStruct((B, T, D), jnp.float32),
        compiler_params=pltpu.CompilerParams(
            dimension_semantics=("parallel", "arbitrary", "arbitrary")),
    )(idx, scl, x16, skill_gate, skill_up, skill_down)


def _shared_body(nf, x_ref, g_ref, u_ref, d_ref, o_ref, acc_ref):
    f = pl.program_id(1)

    @pl.when(f == 0)
    def _():
        acc_ref[...] = jnp.zeros_like(acc_ref)

    xb = x_ref[0]
    g = g_ref[...].astype(jnp.bfloat16)
    u = u_ref[...].astype(jnp.bfloat16)
    d = d_ref[...].astype(jnp.bfloat16)
    xg = jnp.dot(xb, g, preferred_element_type=jnp.float32)
    xu = jnp.dot(xb, u, preferred_element_type=jnp.float32)
    h = jax.nn.gelu(xg, approximate=True) * xu
    acc_ref[...] += jnp.dot(h.astype(jnp.bfloat16), d,
                            preferred_element_type=jnp.float32)

    @pl.when(f == nf - 1)
    def _():
        o_ref[0] = acc_ref[...]


def _shared(x16, shared_gate, shared_up, shared_down, fblk):
    B, T, D = x16.shape
    F = shared_gate.shape[1]
    nf = F // fblk
    grid = (B, nf)
    return pl.pallas_call(
        functools.partial(_shared_body, nf),
        grid=grid,
        in_specs=[
            pl.BlockSpec((1, T, D), lambda s, f: (s, 0, 0)),
            pl.BlockSpec((D, fblk), lambda s, f: (0, f)),
            pl.BlockSpec((D, fblk), lambda s, f: (0, f)),
            pl.BlockSpec((fblk, D), lambda s, f: (f, 0)),
        ],
        out_specs=pl.BlockSpec((1, T, D), lambda s, f: (s, 0, 0)),
        scratch_shapes=[pltpu.VMEM((T, D), jnp.float32)],
        out_shape=jax.ShapeDtypeStruct((B, T, D), jnp.float32),
        compiler_params=pltpu.CompilerParams(
            dimension_semantics=("parallel", "arbitrary")),
    )(x16, shared_gate, shared_up, shared_down)


def _add_body(a_ref, b_ref, o_ref):
    o_ref[...] = a_ref[...] + b_ref[...]


def _add(a, b, tblk):
    B, T, D = a.shape
    grid = (B, T // tblk)
    bs = pl.BlockSpec((1, tblk, D), lambda s, t: (s, t, 0))
    return pl.pallas_call(
        _add_body,
        grid=grid,
        in_specs=[bs, bs],
        out_specs=bs,
        out_shape=jax.ShapeDtypeStruct((B, T, D), jnp.float32),
        compiler_params=pltpu.CompilerParams(
            dimension_semantics=("parallel", "parallel")),
    )(a, b)


def kernel(x, router_logits, skill_gate, skill_up, skill_down, shared_gate,
           shared_up, shared_down):
    B, T, D = x.shape
    F = shared_gate.shape[1]
    fblk = 512 if F % 512 == 0 else F
    tblk = 1024 if T % 1024 == 0 else T

    idx, scl = _route(router_logits)
    x16 = x.astype(jnp.bfloat16)
    shared_out = _shared(x16, shared_gate, shared_up, shared_down, fblk)
    routed_out = _routed(x16, idx, scl, skill_gate, skill_up, skill_down,
                         fblk)
    return _add(routed_out, shared_out, tblk)


# speedup vs baseline: 3.7464x; 1.0884x over previous
import functools

import jax
import jax.numpy as jnp
from jax.experimental import pallas as pl
from jax.experimental.pallas import tpu as pltpu


def _router_body(logits_ref, idx_ref, scl_ref):
    lg = logits_ref[...].astype(jnp.float32)
    m = jnp.max(lg, axis=-1, keepdims=True)
    e = jnp.exp(lg - m)
    p = e / jnp.sum(e, axis=-1, keepdims=True)
    B, E = p.shape
    cols = jax.lax.broadcasted_iota(jnp.int32, (B, E), 1)
    v1 = jnp.max(p, axis=-1)
    i1 = jnp.min(jnp.where(p == v1[:, None], cols, E), axis=-1)
    pm = jnp.where(cols == i1[:, None], -jnp.inf, p)
    v2 = jnp.max(pm, axis=-1)
    i2 = jnp.min(jnp.where(pm == v2[:, None], cols, E), axis=-1)
    tot = v1 + v2
    idx_ref[...] = jnp.stack([i1, i2], axis=-1).astype(jnp.int32)
    scl_ref[...] = jnp.stack([v1 / tot, v2 / tot], axis=-1)


def _route(router_logits):
    B, E = router_logits.shape
    return pl.pallas_call(
        _router_body,
        out_shape=(
            jax.ShapeDtypeStruct((B, 2), jnp.int32),
            jax.ShapeDtypeStruct((B, 2), jnp.float32),
        ),
    )(router_logits)


def _routed_body_weighted(nf, idx_ref, scl_ref, x_ref, g_ref, u_ref, d_ref,
                          o_ref, acc_ref):
    s, j, f = pl.program_id(0), pl.program_id(2), pl.program_id(3)

    @pl.when(jnp.logical_and(j == 0, f == 0))
    def _():
        acc_ref[...] = jnp.zeros_like(acc_ref)

    xb = x_ref[0]
    w = scl_ref[s, j]
    g = g_ref[0].astype(jnp.bfloat16)
    u = u_ref[0].astype(jnp.bfloat16)
    d = (w * d_ref[0]).astype(jnp.bfloat16)
    xg = jnp.dot(xb, g, preferred_element_type=jnp.float32)
    xu = jnp.dot(xb, u, preferred_element_type=jnp.float32)
    h = jax.nn.gelu(xg, approximate=True) * xu
    acc_ref[...] += jnp.dot(h.astype(jnp.bfloat16), d,
                            preferred_element_type=jnp.float32)

    @pl.when(jnp.logical_and(j == 1, f == nf - 1))
    def _():
        o_ref[0] = acc_ref[...]


def _routed(x16, idx, scl, skill_gate, skill_up, skill_down, tblk, fblk):
    B, T, D = x16.shape
    E, _, F = skill_gate.shape
    nf = F // fblk
    nt = T // tblk
    grid = (B, nt, 2, nf)
    spec = pltpu.PrefetchScalarGridSpec(
        num_scalar_prefetch=2,
        grid=grid,
        in_specs=[
            pl.BlockSpec((1, tblk, D), lambda s, t, j, f, i_r, s_r: (s, t, 0)),
            pl.BlockSpec((1, D, fblk),
                         lambda s, t, j, f, i_r, s_r: (i_r[s, j], 0, f)),
            pl.BlockSpec((1, D, fblk),
                         lambda s, t, j, f, i_r, s_r: (i_r[s, j], 0, f)),
            pl.BlockSpec((1, fblk, D),
                         lambda s, t, j, f, i_r, s_r: (i_r[s, j], f, 0)),
        ],
        out_specs=pl.BlockSpec((1, tblk, D),
                               lambda s, t, j, f, i_r, s_r: (s, t, 0)),
        scratch_shapes=[pltpu.VMEM((tblk, D), jnp.float32)],
    )
    return pl.pallas_call(
        functools.partial(_routed_body_weighted, nf),
        grid_spec=spec,
        out_shape=jax.ShapeDtypeStruct((B, T, D), jnp.float32),
        compiler_params=pltpu.CompilerParams(
            dimension_semantics=("parallel", "parallel", "arbitrary",
                                 "arbitrary")),
    )(idx, scl, x16, skill_gate, skill_up, skill_down)


def _shared_body(nf, x_ref, g_ref, u_ref, d_ref, o_ref, acc_ref):
    f = pl.program_id(2)

    @pl.when(f == 0)
    def _():
        acc_ref[...] = jnp.zeros_like(acc_ref)

    xb = x_ref[0]
    g = g_ref[...].astype(jnp.bfloat16)
    u = u_ref[...].astype(jnp.bfloat16)
    d = d_ref[...].astype(jnp.bfloat16)
    xg = jnp.dot(xb, g, preferred_element_type=jnp.float32)
    xu = jnp.dot(xb, u, preferred_element_type=jnp.float32)
    h = jax.nn.gelu(xg, approximate=True) * xu
    acc_ref[...] += jnp.dot(h.astype(jnp.bfloat16), d,
                            preferred_element_type=jnp.float32)

    @pl.when(f == nf - 1)
    def _():
        o_ref[0] = acc_ref[...]


def _shared(x16, shared_gate, shared_up, shared_down, tblk, fblk):
    B, T, D = x16.shape
    F = shared_gate.shape[1]
    nf = F // fblk
    nt = T // tblk
    grid = (B, nt, nf)
    return pl.pallas_call(
        functools.partial(_shared_body, nf),
        grid=grid,
        in_specs=[
            pl.BlockSpec((1, tblk, D), lambda s, t, f: (s, t, 0)),
            pl.BlockSpec((D, fblk), lambda s, t, f: (0, f)),
            pl.BlockSpec((D, fblk), lambda s, t, f: (0, f)),
            pl.BlockSpec((fblk, D), lambda s, t, f: (f, 0)),
        ],
        out_specs=pl.BlockSpec((1, tblk, D), lambda s, t, f: (s, t, 0)),
        scratch_shapes=[pltpu.VMEM((tblk, D), jnp.float32)],
        out_shape=jax.ShapeDtypeStruct((B, T, D), jnp.float32),
        compiler_params=pltpu.CompilerParams(
            dimension_semantics=("parallel", "parallel", "arbitrary")),
    )(x16, shared_gate, shared_up, shared_down)


def _add_body(a_ref, b_ref, o_ref):
    o_ref[...] = a_ref[...] + b_ref[...]


def _add(a, b, tblk):
    B, T, D = a.shape
    grid = (B, T // tblk)
    bs = pl.BlockSpec((1, tblk, D), lambda s, t: (s, t, 0))
    return pl.pallas_call(
        _add_body,
        grid=grid,
        in_specs=[bs, bs],
        out_specs=bs,
        out_shape=jax.ShapeDtypeStruct((B, T, D), jnp.float32),
        compiler_params=pltpu.CompilerParams(
            dimension_semantics=("parallel", "parallel")),
    )(a, b)


def kernel(x, router_logits, skill_gate, skill_up, skill_down, shared_gate,
           shared_up, shared_down):
    B, T, D = x.shape
    F = shared_gate.shape[1]
    fblk = 1024 if F % 1024 == 0 else F
    tblk = 1024 if T % 1024 == 0 else T

    idx, scl = _route(router_logits)
    x16 = x.astype(jnp.bfloat16)
    shared_out = _shared(x16, shared_gate, shared_up, shared_down, tblk, fblk)
    routed_out = _routed(x16, idx, scl, skill_gate, skill_up, skill_down,
                         tblk, fblk)
    return _add(routed_out, shared_out, tblk)


# speedup vs baseline: 3.9103x; 1.0437x over previous
import functools

import jax
import jax.numpy as jnp
from jax.experimental import pallas as pl
from jax.experimental.pallas import tpu as pltpu


def _router_body(logits_ref, idx_ref, scl_ref):
    lg = logits_ref[...].astype(jnp.float32)
    m = jnp.max(lg, axis=-1, keepdims=True)
    e = jnp.exp(lg - m)
    p = e / jnp.sum(e, axis=-1, keepdims=True)
    B, E = p.shape
    cols = jax.lax.broadcasted_iota(jnp.int32, (B, E), 1)
    v1 = jnp.max(p, axis=-1)
    i1 = jnp.min(jnp.where(p == v1[:, None], cols, E), axis=-1)
    pm = jnp.where(cols == i1[:, None], -jnp.inf, p)
    v2 = jnp.max(pm, axis=-1)
    i2 = jnp.min(jnp.where(pm == v2[:, None], cols, E), axis=-1)
    tot = v1 + v2
    idx_ref[...] = jnp.stack([i1, i2], axis=-1).astype(jnp.int32)
    scl_ref[...] = jnp.stack([v1 / tot, v2 / tot], axis=-1)


def _route(router_logits):
    B, E = router_logits.shape
    return pl.pallas_call(
        _router_body,
        out_shape=(
            jax.ShapeDtypeStruct((B, 2), jnp.int32),
            jax.ShapeDtypeStruct((B, 2), jnp.float32),
        ),
    )(router_logits)


def _routed_body_weighted(nf, idx_ref, scl_ref, x_ref, prev_ref, g_ref,
                          u_ref, d_ref, o_ref, acc_ref):
    s, j, f = pl.program_id(0), pl.program_id(2), pl.program_id(3)

    @pl.when(jnp.logical_and(j == 0, f == 0))
    def _():
        acc_ref[...] = prev_ref[0]

    xb = x_ref[0]
    w = scl_ref[s, j]
    g = g_ref[0].astype(jnp.bfloat16)
    u = u_ref[0].astype(jnp.bfloat16)
    d = (w * d_ref[0]).astype(jnp.bfloat16)
    xg = jnp.dot(xb, g, preferred_element_type=jnp.float32)
    xu = jnp.dot(xb, u, preferred_element_type=jnp.float32)
    h = jax.nn.gelu(xg, approximate=True) * xu
    acc_ref[...] += jnp.dot(h.astype(jnp.bfloat16), d,
                            preferred_element_type=jnp.float32)

    @pl.when(jnp.logical_and(j == 1, f == nf - 1))
    def _():
        o_ref[0] = acc_ref[...]


def _routed(x16, idx, scl, skill_gate, skill_up, skill_down, prev, tblk,
            fblk):
    B, T, D = x16.shape
    E, _, F = skill_gate.shape
    nf = F // fblk
    nt = T // tblk
    grid = (B, nt, 2, nf)
    spec = pltpu.PrefetchScalarGridSpec(
        num_scalar_prefetch=2,
        grid=grid,
        in_specs=[
            pl.BlockSpec((1, tblk, D), lambda s, t, j, f, i_r, s_r: (s, t, 0)),
            pl.BlockSpec((1, tblk, D), lambda s, t, j, f, i_r, s_r: (s, t, 0)),
            pl.BlockSpec((1, D, fblk),
                         lambda s, t, j, f, i_r, s_r: (i_r[s, j], 0, f)),
            pl.BlockSpec((1, D, fblk),
                         lambda s, t, j, f, i_r, s_r: (i_r[s, j], 0, f)),
            pl.BlockSpec((1, fblk, D),
                         lambda s, t, j, f, i_r, s_r: (i_r[s, j], f, 0)),
        ],
        out_specs=pl.BlockSpec((1, tblk, D),
                               lambda s, t, j, f, i_r, s_r: (s, t, 0)),
        scratch_shapes=[pltpu.VMEM((tblk, D), jnp.float32)],
    )
    return pl.pallas_call(
        functools.partial(_routed_body_weighted, nf),
        grid_spec=spec,
        out_shape=jax.ShapeDtypeStruct((B, T, D), jnp.float32),
        compiler_params=pltpu.CompilerParams(
            dimension_semantics=("parallel", "parallel", "arbitrary",
                                 "arbitrary")),
    )(idx, scl, x16, prev, skill_gate, skill_up, skill_down)


def _shared_body(nf, x_ref, g_ref, u_ref, d_ref, o_ref, acc_ref):
    f = pl.program_id(2)

    @pl.when(f == 0)
    def _():
        acc_ref[...] = jnp.zeros_like(acc_ref)

    xb = x_ref[0]
    g = g_ref[...].astype(jnp.bfloat16)
    u = u_ref[...].astype(jnp.bfloat16)
    d = d_ref[...].astype(jnp.bfloat16)
    xg = jnp.dot(xb, g, preferred_element_type=jnp.float32)
    xu = jnp.dot(xb, u, preferred_element_type=jnp.float32)
    h = jax.nn.gelu(xg, approximate=True) * xu
    acc_ref[...] += jnp.dot(h.astype(jnp.bfloat16), d,
                            preferred_element_type=jnp.float32)

    @pl.when(f == nf - 1)
    def _():
        o_ref[0] = acc_ref[...]


def _shared(x16, shared_gate, shared_up, shared_down, tblk, fblk):
    B, T, D = x16.shape
    F = shared_gate.shape[1]
    nf = F // fblk
    nt = T // tblk
    grid = (B, nt, nf)
    return pl.pallas_call(
        functools.partial(_shared_body, nf),
        grid=grid,
        in_specs=[
            pl.BlockSpec((1, tblk, D), lambda s, t, f: (s, t, 0)),
            pl.BlockSpec((D, fblk), lambda s, t, f: (0, f)),
            pl.BlockSpec((D, fblk), lambda s, t, f: (0, f)),
            pl.BlockSpec((fblk, D), lambda s, t, f: (f, 0)),
        ],
        out_specs=pl.BlockSpec((1, tblk, D), lambda s, t, f: (s, t, 0)),
        scratch_shapes=[pltpu.VMEM((tblk, D), jnp.float32)],
        out_shape=jax.ShapeDtypeStruct((B, T, D), jnp.float32),
        compiler_params=pltpu.CompilerParams(
            dimension_semantics=("parallel", "parallel", "arbitrary")),
    )(x16, shared_gate, shared_up, shared_down)


def kernel(x, router_logits, skill_gate, skill_up, skill_down, shared_gate,
           shared_up, shared_down):
    B, T, D = x.shape
    F = shared_gate.shape[1]
    fblk = 1024 if F % 1024 == 0 else F
    tblk = 1024 if T % 1024 == 0 else T

    idx, scl = _route(router_logits)
    x16 = x.astype(jnp.bfloat16)
    shared_out = _shared(x16, shared_gate, shared_up, shared_down, tblk, fblk)
    return _routed(x16, idx, scl, skill_gate, skill_up, skill_down,
                   shared_out, tblk, fblk)


# speedup vs baseline: 3.9199x; 1.0025x over previous
import functools

import jax
import jax.numpy as jnp
from jax.experimental import pallas as pl
from jax.experimental.pallas import tpu as pltpu


def _router_body(logits_ref, idx_ref, scl_ref):
    lg = logits_ref[...].astype(jnp.float32)
    m = jnp.max(lg, axis=-1, keepdims=True)
    e = jnp.exp(lg - m)
    p = e / jnp.sum(e, axis=-1, keepdims=True)
    B, E = p.shape
    cols = jax.lax.broadcasted_iota(jnp.int32, (B, E), 1)
    v1 = jnp.max(p, axis=-1)
    i1 = jnp.min(jnp.where(p == v1[:, None], cols, E), axis=-1)
    pm = jnp.where(cols == i1[:, None], -jnp.inf, p)
    v2 = jnp.max(pm, axis=-1)
    i2 = jnp.min(jnp.where(pm == v2[:, None], cols, E), axis=-1)
    tot = v1 + v2
    idx_ref[...] = jnp.stack([i1, i2], axis=-1).astype(jnp.int32)
    scl_ref[...] = jnp.stack([v1 / tot, v2 / tot], axis=-1)


def _route(router_logits):
    B, E = router_logits.shape
    return pl.pallas_call(
        _router_body,
        out_shape=(
            jax.ShapeDtypeStruct((B, 2), jnp.int32),
            jax.ShapeDtypeStruct((B, 2), jnp.float32),
        ),
    )(router_logits)


def _routed_body_weighted(nf, idx_ref, scl_ref, x_ref, prev_ref, g_ref,
                          u_ref, d_ref, o_ref, acc_ref):
    s, j, f = pl.program_id(0), pl.program_id(2), pl.program_id(3)

    @pl.when(jnp.logical_and(j == 0, f == 0))
    def _():
        acc_ref[...] = prev_ref[0].astype(jnp.float32)

    xb = x_ref[0]
    w = scl_ref[s, j]
    g = g_ref[0].astype(jnp.bfloat16)
    u = u_ref[0].astype(jnp.bfloat16)
    d = (w * d_ref[0]).astype(jnp.bfloat16)
    xg = jnp.dot(xb, g, preferred_element_type=jnp.float32)
    xu = jnp.dot(xb, u, preferred_element_type=jnp.float32)
    h = jax.nn.gelu(xg, approximate=True) * xu
    acc_ref[...] += jnp.dot(h.astype(jnp.bfloat16), d,
                            preferred_element_type=jnp.float32)

    @pl.when(jnp.logical_and(j == 1, f == nf - 1))
    def _():
        o_ref[0] = acc_ref[...]


def _routed(x16, idx, scl, skill_gate, skill_up, skill_down, prev, tblk,
            fblk):
    B, T, D = x16.shape
    E, _, F = skill_gate.shape
    nf = F // fblk
    nt = T // tblk
    grid = (B, nt, 2, nf)
    spec = pltpu.PrefetchScalarGridSpec(
        num_scalar_prefetch=2,
        grid=grid,
        in_specs=[
            pl.BlockSpec((1, tblk, D), lambda s, t, j, f, i_r, s_r: (s, t, 0)),
            pl.BlockSpec((1, tblk, D), lambda s, t, j, f, i_r, s_r: (s, t, 0)),
            pl.BlockSpec((1, D, fblk),
                         lambda s, t, j, f, i_r, s_r: (i_r[s, j], 0, f)),
            pl.BlockSpec((1, D, fblk),
                         lambda s, t, j, f, i_r, s_r: (i_r[s, j], 0, f)),
            pl.BlockSpec((1, fblk, D),
                         lambda s, t, j, f, i_r, s_r: (i_r[s, j], f, 0)),
        ],
        out_specs=pl.BlockSpec((1, tblk, D),
                               lambda s, t, j, f, i_r, s_r: (s, t, 0)),
        scratch_shapes=[pltpu.VMEM((tblk, D), jnp.float32)],
    )
    return pl.pallas_call(
        functools.partial(_routed_body_weighted, nf),
        grid_spec=spec,
        out_shape=jax.ShapeDtypeStruct((B, T, D), jnp.float32),
        compiler_params=pltpu.CompilerParams(
            dimension_semantics=("parallel", "parallel", "arbitrary",
                                 "arbitrary")),
    )(idx, scl, x16, prev, skill_gate, skill_up, skill_down)


def _shared_body(nf, x_ref, g_ref, u_ref, d_ref, o_ref, acc_ref):
    f = pl.program_id(2)

    @pl.when(f == 0)
    def _():
        acc_ref[...] = jnp.zeros_like(acc_ref)

    xb = x_ref[0]
    g = g_ref[...].astype(jnp.bfloat16)
    u = u_ref[...].astype(jnp.bfloat16)
    d = d_ref[...].astype(jnp.bfloat16)
    xg = jnp.dot(xb, g, preferred_element_type=jnp.float32)
    xu = jnp.dot(xb, u, preferred_element_type=jnp.float32)
    h = jax.nn.gelu(xg, approximate=True) * xu
    acc_ref[...] += jnp.dot(h.astype(jnp.bfloat16), d,
                            preferred_element_type=jnp.float32)

    @pl.when(f == nf - 1)
    def _():
        o_ref[0] = acc_ref[...].astype(jnp.bfloat16)


def _shared(x16, shared_gate, shared_up, shared_down, tblk, fblk):
    B, T, D = x16.shape
    F = shared_gate.shape[1]
    nf = F // fblk
    nt = T // tblk
    grid = (B, nt, nf)
    return pl.pallas_call(
        functools.partial(_shared_body, nf),
        grid=grid,
        in_specs=[
            pl.BlockSpec((1, tblk, D), lambda s, t, f: (s, t, 0)),
            pl.BlockSpec((D, fblk), lambda s, t, f: (0, f)),
            pl.BlockSpec((D, fblk), lambda s, t, f: (0, f)),
            pl.BlockSpec((fblk, D), lambda s, t, f: (f, 0)),
        ],
        out_specs=pl.BlockSpec((1, tblk, D), lambda s, t, f: (s, t, 0)),
        scratch_shapes=[pltpu.VMEM((tblk, D), jnp.float32)],
        out_shape=jax.ShapeDtypeStruct((B, T, D), jnp.bfloat16),
        compiler_params=pltpu.CompilerParams(
            dimension_semantics=("parallel", "parallel", "arbitrary")),
    )(x16, shared_gate, shared_up, shared_down)


def kernel(x, router_logits, skill_gate, skill_up, skill_down, shared_gate,
           shared_up, shared_down):
    B, T, D = x.shape
    F = shared_gate.shape[1]
    fblk = 1024 if F % 1024 == 0 else F
    tblk = 1024 if T % 1024 == 0 else T

    idx, scl = _route(router_logits)
    x16 = x.astype(jnp.bfloat16)
    shared_out = _shared(x16, shared_gate, shared_up, shared_down, tblk, fblk)
    return _routed(x16, idx, scl, skill_gate, skill_up, skill_down,
                   shared_out, tblk, fblk)
